# Initial kernel scaffold; baseline (speedup 1.0000x reference)
import jax
import jax.numpy as jnp
from jax.experimental import pallas as pl


def kernel(h, edges, W_lin, W_attn):
    raise NotImplementedError("write your pallas kernel here")


# speedup vs baseline: 36.1435x; 36.1435x over previous
import dataclasses
import functools

import jax
import jax.numpy as jnp
from jax import lax
from jax.experimental import pallas as pl
from jax.experimental.pallas import tpu as pltpu
from jax.experimental.pallas import tpu_sc as plsc

N_NODES = 10000
N_EDGES = 320000
K = 8
D_K = 16

NC = 2
NS = 16
NW = NC * NS
L = 16

EW = N_EDGES // NW
CHUNK = 2000
NCHUNK = EW // CHUNK
GSUB = 80
NGSUB = CHUNK // GSUB
NGROUP = CHUNK // L


def _tc_table_body(h_ref, w_ref, o_ref):
    o_ref[...] = jnp.dot(h_ref[...], w_ref[...],
                         preferred_element_type=jnp.float32)


def _full(v):
    return jnp.full((L,), v, jnp.int32)


def _sc_edge_body(t_hbm, src_hbm, dst_hbm, out_hbm,
                  src_v, dst_v, rows_s, rows_d, out_v, sem, sem2):
    wid = lax.axis_index("c") * NS + lax.axis_index("s")
    base = wid * EW
    lane = lax.iota(jnp.int32, L)

    @pl.loop(0, NCHUNK)
    def _chunk(t):
        off = base + t * CHUNK
        pltpu.sync_copy(src_hbm.at[pl.ds(off, CHUNK)], src_v)
        pltpu.sync_copy(dst_hbm.at[pl.ds(off, CHUNK)], dst_v)
        cs = pltpu.async_copy(t_hbm.at[src_v], rows_s, sem)
        cd = pltpu.async_copy(t_hbm.at[dst_v], rows_d, sem2)
        cs.wait()
        cd.wait()

        @pl.loop(0, NGROUP)
        def _group(g):
            rows = g * L + lane
            a_s = [plsc.load_gather(rows_s, [rows, _full(i)])
                   for i in range(K)]
            a_d = [plsc.load_gather(rows_d, [rows, _full(K + j)])
                   for j in range(K)]
            scores = []
            for j in range(K):
                acc = None
                for i in range(K):
                    s = a_s[i] + a_d[j]
                    e = jnp.exp(jnp.maximum(s, 0.01 * s))
                    acc = e if acc is None else acc + e
                scores.append(acc)
            total = scores[0]
            for j in range(1, K):
                total = total + scores[j]
            inv = 1.0 / total
            for j in range(K):
                plsc.store_scatter(out_v, [rows, _full(j)], scores[j] * inv)

        pltpu.sync_copy(out_v, out_hbm.at[pl.ds(off, CHUNK)])


@jax.jit
def kernel(h, edges, W_lin, W_attn):
    edges = edges.astype(jnp.int32)
    src = edges[:, 0]
    dst = edges[:, 1]

    ws = W_attn[0, :D_K] @ W_lin
    wd = W_attn[0, D_K:] @ W_lin
    eye = jnp.eye(K, dtype=jnp.float32)
    W2 = jnp.concatenate([jnp.kron(eye, ws[:, None]),
                          jnp.kron(eye, wd[:, None])], axis=1)

    table = pl.pallas_call(
        _tc_table_body,
        out_shape=jax.ShapeDtypeStruct((N_NODES, 2 * K), jnp.float32),
    )(h.reshape(N_NODES, K * D_K), W2)

    cp = pltpu.CompilerParams(use_tc_tiling_on_sc=False)
    if "needs_layout_passes" in pltpu.CompilerParams.__dataclass_fields__:
        cp = dataclasses.replace(cp, needs_layout_passes=False)
    sc_kernel = pl.kernel(
        _sc_edge_body,
        out_type=jax.ShapeDtypeStruct((N_EDGES, K), jnp.float32),
        mesh=plsc.VectorSubcoreMesh(core_axis_name="c", subcore_axis_name="s"),
        compiler_params=cp,
        scratch_types=[
            pltpu.VMEM((CHUNK,), jnp.int32),
            pltpu.VMEM((CHUNK,), jnp.int32),
            pltpu.VMEM((CHUNK, 2 * K), jnp.float32),
            pltpu.VMEM((CHUNK, 2 * K), jnp.float32),
            pltpu.VMEM((CHUNK, K), jnp.float32),
            pltpu.SemaphoreType.DMA,
            pltpu.SemaphoreType.DMA,
        ],
    )
    attn = sc_kernel(table, src, dst)
    return attn[:, :, None]
